# Initial kernel scaffold; baseline (speedup 1.0000x reference)
import jax
import jax.numpy as jnp
from jax.experimental import pallas as pl


def kernel(x, w1t, b1, w2t, b2):
    raise NotImplementedError("write your pallas kernel here")


# speedup vs baseline: 1.0001x; 1.0001x over previous
import functools

import jax
import jax.numpy as jnp
from jax.experimental import pallas as pl
from jax.experimental.pallas import tpu as pltpu


def _ae_fused(x_ref, w1t_ref, b1_ref, w2t_ref, b2_ref, enc_ref, dec_ref):
    h = jnp.dot(x_ref[...], w1t_ref[...], preferred_element_type=jnp.float32)
    enc_ref[...] = jnp.maximum(h + b1_ref[...], 0.0)
    d = jnp.dot(enc_ref[...], w2t_ref[...], preferred_element_type=jnp.float32)
    dec_ref[...] = d + b2_ref[...]


@functools.partial(jax.jit, static_argnames=("bt",))
def _ae_call(x, w1t, b1, w2t, b2, *, bt):
    B, nb_param = x.shape
    hidden = w1t.shape[1]
    bt = min(bt, B)
    grid = (pl.cdiv(B, bt),)

    return pl.pallas_call(
        _ae_fused,
        out_shape=(
            jax.ShapeDtypeStruct((B, hidden), x.dtype),
            jax.ShapeDtypeStruct((B, nb_param), x.dtype),
        ),
        grid_spec=pl.GridSpec(
            grid=grid,
            in_specs=[
                pl.BlockSpec((bt, nb_param), lambda i: (i, 0)),
                pl.BlockSpec((nb_param, hidden), lambda i: (0, 0)),
                pl.BlockSpec((1, hidden), lambda i: (0, 0)),
                pl.BlockSpec((hidden, nb_param), lambda i: (0, 0)),
                pl.BlockSpec((1, nb_param), lambda i: (0, 0)),
            ],
            out_specs=[
                pl.BlockSpec((bt, hidden), lambda i: (i, 0)),
                pl.BlockSpec((bt, nb_param), lambda i: (i, 0)),
            ],
        ),
        compiler_params=pltpu.CompilerParams(
            dimension_semantics=("parallel",),
            vmem_limit_bytes=64 * 1024 * 1024,
        ),
    )(x, w1t, b1, w2t, b2)


def kernel(x, w1t, b1, w2t, b2):
    return _ae_call(x, w1t, b1, w2t, b2, bt=512)


# speedup vs baseline: 1.0159x; 1.0158x over previous
import functools

import jax
import jax.numpy as jnp
from jax.experimental import pallas as pl
from jax.experimental.pallas import tpu as pltpu


def _ae_fused(x_ref, w1t_ref, b1_ref, w2t_ref, b2_ref, enc_ref, dec_ref):
    h = jnp.dot(x_ref[...], w1t_ref[...], preferred_element_type=jnp.float32)
    enc_ref[...] = jnp.maximum(h + b1_ref[...], 0.0)
    d = jnp.dot(enc_ref[...], w2t_ref[...], preferred_element_type=jnp.float32)
    dec_ref[...] = d + b2_ref[...]


@functools.partial(jax.jit, static_argnames=("bt",))
def _ae_call(x, w1t, b1, w2t, b2, *, bt):
    B, nb_param = x.shape
    hidden = w1t.shape[1]
    bt = min(bt, B)
    grid = (pl.cdiv(B, bt),)

    return pl.pallas_call(
        _ae_fused,
        out_shape=(
            jax.ShapeDtypeStruct((B, hidden), x.dtype),
            jax.ShapeDtypeStruct((B, nb_param), x.dtype),
        ),
        grid_spec=pl.GridSpec(
            grid=grid,
            in_specs=[
                pl.BlockSpec((bt, nb_param), lambda i: (i, 0)),
                pl.BlockSpec((nb_param, hidden), lambda i: (0, 0)),
                pl.BlockSpec((1, hidden), lambda i: (0, 0)),
                pl.BlockSpec((hidden, nb_param), lambda i: (0, 0)),
                pl.BlockSpec((1, nb_param), lambda i: (0, 0)),
            ],
            out_specs=[
                pl.BlockSpec((bt, hidden), lambda i: (i, 0)),
                pl.BlockSpec((bt, nb_param), lambda i: (i, 0)),
            ],
        ),
        compiler_params=pltpu.CompilerParams(
            dimension_semantics=("parallel",),
            vmem_limit_bytes=64 * 1024 * 1024,
        ),
    )(x, w1t, b1, w2t, b2)


def kernel(x, w1t, b1, w2t, b2):
    return _ae_call(x, w1t, b1, w2t, b2, bt=1024)
